# Initial kernel scaffold; baseline (speedup 1.0000x reference)
import jax
import jax.numpy as jnp
from jax.experimental import pallas as pl


def kernel(x, edge_index1, edge_index2, W1l, W1r, b1, W2l, W2r, b2):
    raise NotImplementedError("write your pallas kernel here")


# speedup vs baseline: 8.3865x; 8.3865x over previous
import functools

import jax
import jax.numpy as jnp
from jax import lax
from jax.experimental import pallas as pl
from jax.experimental.pallas import tpu as pltpu
from jax.experimental.pallas import tpu_sc as plsc

N0, N1, N2 = 10000, 4000, 1000
E1, E2 = 320000, 64000
D = 128
DP = 144
NC, NS = 2, 16
NW = NC * NS
OUT_PAD = 1024


def _sc_aggregate(n_rows, n_seg, n_chunks, chunk):
  mesh = plsc.VectorSubcoreMesh(
      core_axis_name="c", subcore_axis_name="s", num_cores=NC,
      num_subcores=NS)
  zrows = n_seg // NS
  orows = OUT_PAD // NS
  zb = 64

  @functools.partial(
      pl.kernel,
      out_type=jax.ShapeDtypeStruct((NC, OUT_PAD, DP), jnp.float32),
      mesh=mesh,
      scratch_types=[
          pltpu.VMEM((n_chunks, chunk), jnp.int32),
          pltpu.VMEM((n_chunks, chunk), jnp.int32),
          pltpu.VMEM((chunk, DP), jnp.float32),
          pltpu.VMEM((zb, DP), jnp.float32),
          pltpu.VMEM_SHARED((n_seg, DP), jnp.float32),
          pltpu.SemaphoreType.DMA,
      ],
      compiler_params=pltpu.CompilerParams(use_tc_tiling_on_sc=False),
  )
  def agg(table_hbm, src_hbm, dst_hbm, out_hbm, src_v, dst_v, rows_v, zbuf,
          acc, sem):
    cid = lax.axis_index("c")
    sid = lax.axis_index("s")
    wid = cid * NS + sid

    zero = jnp.zeros((16,), jnp.float32)
    for r in range(zb):
      for k in range(DP // 16):
        zbuf[r, pl.ds(k * 16, 16)] = zero
    base = sid * zrows
    for k in range(zrows // zb):
      pltpu.sync_copy(zbuf, acc.at[pl.ds(base + k * zb, zb)])
    rem = zrows % zb
    if rem:
      pltpu.sync_copy(zbuf.at[pl.ds(0, rem)],
                      acc.at[pl.ds(base + (zrows // zb) * zb, rem)])

    pltpu.sync_copy(src_hbm.at[wid], src_v)
    pltpu.sync_copy(dst_hbm.at[wid], dst_v)
    plsc.subcore_barrier()

    def body(c, carry):
      pltpu.async_copy(table_hbm.at[src_v.at[c]], rows_v, sem).wait()
      pltpu.sync_copy(rows_v, acc.at[dst_v.at[c]], add=True)
      return carry

    lax.fori_loop(0, n_chunks, body, 0)
    plsc.subcore_barrier()
    pltpu.sync_copy(acc.at[pl.ds(sid * orows, orows)],
                    out_hbm.at[cid, pl.ds(sid * orows, orows)])

  return agg


def _tc_layer1(p, x1, W1l, W1r, b1):
  def body(p_ref, x_ref, wl_ref, wr_ref, b_ref, out_ref):
    ps = p_ref[0] + p_ref[1]
    s = ps[:, :D]
    cnt = ps[:, D:D + 1]
    mean = s / jnp.maximum(cnt, 1.0)
    h = mean @ wl_ref[...] + x_ref[...] @ wr_ref[...] + b_ref[...]
    h = jnp.maximum(h, 0.0)
    out_ref[...] = jnp.concatenate(
        [h, jnp.ones((h.shape[0], 1), jnp.float32),
         jnp.zeros((h.shape[0], DP - D - 1), jnp.float32)], axis=1)

  return pl.pallas_call(
      body,
      out_shape=jax.ShapeDtypeStruct((N2, DP), jnp.float32),
  )(p, x1, W1l, W1r, b1.reshape(1, D))


def _tc_layer2(p, hpad, W2l, W2r, b2):
  def body(p_ref, h_ref, wl_ref, wr_ref, b_ref, out_ref):
    ps = p_ref[0] + p_ref[1]
    s = ps[:, :D]
    cnt = ps[:, D:D + 1]
    mean = s / jnp.maximum(cnt, 1.0)
    h = h_ref[:, :D]
    z = mean @ wl_ref[...] + h @ wr_ref[...] + b_ref[...]
    m = jnp.max(z, axis=-1, keepdims=True)
    e = jnp.exp(z - m)
    lse = jnp.log(jnp.sum(e, axis=-1, keepdims=True)) + m
    out_ref[...] = z - lse

  return pl.pallas_call(
      body,
      out_shape=jax.ShapeDtypeStruct((N2, D), jnp.float32),
  )(p, hpad, W2l, W2r, b2.reshape(1, D))


def kernel(x, edge_index1, edge_index2, W1l, W1r, b1, W2l, W2r, b2):
  x4 = x[:N1]
  pad = jnp.concatenate(
      [jnp.ones((N1, 1), jnp.float32),
       jnp.zeros((N1, DP - D - 1), jnp.float32)], axis=1)
  xpad = jnp.concatenate([x4, pad], axis=1)

  ch1, nch1 = 80, E1 // NW // 80
  ch2, nch2 = 80, E2 // NW // 80
  e1s = edge_index1[0].reshape(NW, nch1, ch1)
  e1d = edge_index1[1].reshape(NW, nch1, ch1)
  e2s = edge_index2[0].reshape(NW, nch2, ch2)
  e2d = edge_index2[1].reshape(NW, nch2, ch2)

  p1 = _sc_aggregate(N1, 4096, nch1, ch1)(xpad, e1s, e1d)
  hpad = _tc_layer1(p1[:, :N2], x[:N2], W1l, W1r, b1)
  p2 = _sc_aggregate(N2, OUT_PAD, nch2, ch2)(hpad, e2s, e2d)
  return _tc_layer2(p2[:, :N2], hpad, W2l, W2r, b2)


# speedup vs baseline: 10.0715x; 1.2009x over previous
import functools

import jax
import jax.numpy as jnp
from jax import lax
from jax.experimental import pallas as pl
from jax.experimental.pallas import tpu as pltpu
from jax.experimental.pallas import tpu_sc as plsc

N0, N1, N2 = 10000, 4000, 1000
E1, E2 = 320000, 64000
D = 128
DP = 144
NC, NS = 2, 16
NW = NC * NS
NSEG = 1024
CH = 128


def _sc_aggregate(n_rows, epw):
  mesh = plsc.VectorSubcoreMesh(
      core_axis_name="c", subcore_axis_name="s", num_cores=NC,
      num_subcores=NS)
  zrows = NSEG // NS
  qcap = epw + CH

  @functools.partial(
      pl.kernel,
      out_type=jax.ShapeDtypeStruct((NC, NSEG, DP), jnp.float32),
      mesh=mesh,
      scratch_types=[
          pltpu.VMEM((epw,), jnp.int32),
          pltpu.VMEM((epw,), jnp.int32),
          pltpu.VMEM((qcap,), jnp.int32),
          pltpu.VMEM((qcap,), jnp.int32),
          pltpu.VMEM((2, CH), jnp.int32),
          pltpu.VMEM((2, CH), jnp.int32),
          pltpu.VMEM((2, CH, DP), jnp.float32),
          pltpu.VMEM((zrows, DP), jnp.float32),
          pltpu.VMEM_SHARED((NSEG, DP), jnp.float32),
          pltpu.SemaphoreType.DMA((2,)),
      ],
      compiler_params=pltpu.CompilerParams(use_tc_tiling_on_sc=False,
                                           needs_layout_passes=False),
  )
  def agg(table_hbm, edges_hbm, out_hbm, src_v, dst_v, qs, qd, qs2, qd2,
          rows, zbuf, acc, semg):
    cid = lax.axis_index("c")
    sid = lax.axis_index("s")
    wid = cid * NS + sid

    zero = jnp.zeros((16,), jnp.float32)
    for r in range(zrows):
      for k in range(DP // 16):
        zbuf[r, pl.ds(k * 16, 16)] = zero
    pltpu.sync_copy(zbuf, acc.at[pl.ds(sid * zrows, zrows)])

    pltpu.sync_copy(edges_hbm.at[0, pl.ds(wid * epw, epw)], src_v)
    pltpu.sync_copy(edges_hbm.at[1, pl.ds(wid * epw, epw)], dst_v)

    iota = lax.iota(jnp.int32, 16)

    def scan_body(i, qn):
      off = pl.multiple_of(i * 16, 16)
      s = src_v[pl.ds(off, 16)]
      d = dst_v[pl.ds(off, 16)]
      m = d < NSEG
      mi = m.astype(jnp.int32)
      pos = qn + plsc.cumsum(mi) - 1
      plsc.store_scatter(qs, [pos], s, mask=m)
      plsc.store_scatter(qd, [pos], d, mask=m)
      return qn + jnp.sum(mi)

    qn = lax.fori_loop(0, epw // 16, scan_body, 0)
    for k in range(CH // 16):
      pos = qn + k * 16 + iota
      plsc.store_scatter(qs, [pos], jnp.zeros((16,), jnp.int32))
      plsc.store_scatter(qd, [pos], jnp.full((16,), NSEG - 8, jnp.int32))
    nch = qn // CH + 1
    plsc.subcore_barrier()

    def stage(c, p):
      for k in range(CH // 16):
        pos = c * CH + k * 16 + iota
        qs2[p, pl.ds(k * 16, 16)] = plsc.load_gather(qs, [pos])
        qd2[p, pl.ds(k * 16, 16)] = plsc.load_gather(qd, [pos])

    def gather(p):
      return pltpu.make_async_copy(
          table_hbm.at[qs2.at[p]], rows.at[p], semg.at[p])

    stage(0, 0)
    gather(0).start()

    def body(c, carry):
      p = c % 2
      o = (c + 1) % 2
      stage(c + 1, o)
      gather(o).start()
      gather(p).wait()
      pltpu.sync_copy(rows.at[p], acc.at[qd2.at[p]], add=True)
      return carry

    lax.fori_loop(0, nch - 1, body, 0)
    last = (nch - 1) % 2
    gather(last).wait()
    pltpu.sync_copy(rows.at[last], acc.at[qd2.at[last]], add=True)

    plsc.subcore_barrier()
    pltpu.sync_copy(acc.at[pl.ds(sid * zrows, zrows)],
                    out_hbm.at[cid, pl.ds(sid * zrows, zrows)])

  return agg


def _tc_layer1(p, x1, W1l, W1r, b1):
  def body(p_ref, x_ref, wl_ref, wr_ref, b_ref, out_ref):
    ps = p_ref[0, :N2] + p_ref[1, :N2]
    s = ps[:, :D]
    cnt = ps[:, D:D + 1]
    mean = s / jnp.maximum(cnt, 1.0)
    h = mean @ wl_ref[...] + x_ref[...] @ wr_ref[...] + b_ref[...]
    h = jnp.maximum(h, 0.0)
    out_ref[...] = jnp.concatenate(
        [h, jnp.ones((N2, 1), jnp.float32),
         jnp.zeros((N2, DP - D - 1), jnp.float32)], axis=1)

  return pl.pallas_call(
      body,
      out_shape=jax.ShapeDtypeStruct((N2, DP), jnp.float32),
  )(p, x1, W1l, W1r, b1.reshape(1, D))


def _tc_layer2(p, hpad, W2l, W2r, b2):
  def body(p_ref, h_ref, wl_ref, wr_ref, b_ref, out_ref):
    ps = p_ref[0, :N2] + p_ref[1, :N2]
    s = ps[:, :D]
    cnt = ps[:, D:D + 1]
    mean = s / jnp.maximum(cnt, 1.0)
    h = h_ref[:, :D]
    z = mean @ wl_ref[...] + h @ wr_ref[...] + b_ref[...]
    m = jnp.max(z, axis=-1, keepdims=True)
    e = jnp.exp(z - m)
    lse = jnp.log(jnp.sum(e, axis=-1, keepdims=True)) + m
    out_ref[...] = z - lse

  return pl.pallas_call(
      body,
      out_shape=jax.ShapeDtypeStruct((N2, D), jnp.float32),
  )(p, hpad, W2l, W2r, b2.reshape(1, D))


def kernel(x, edge_index1, edge_index2, W1l, W1r, b1, W2l, W2r, b2):
  x4 = x[:N1]
  pad = jnp.concatenate(
      [jnp.ones((N1, 1), jnp.float32),
       jnp.zeros((N1, DP - D - 1), jnp.float32)], axis=1)
  xpad = jnp.concatenate([x4, pad], axis=1)

  p1 = _sc_aggregate(N1, E1 // NW)(xpad, edge_index1)
  hpad = _tc_layer1(p1, x[:N2], W1l, W1r, b1)
  p2 = _sc_aggregate(N2, E2 // NW)(hpad, edge_index2)
  return _tc_layer2(p2, hpad, W2l, W2r, b2)


# speedup vs baseline: 21.5332x; 2.1380x over previous
import functools

import jax
import jax.numpy as jnp
from jax import lax
from jax.experimental import pallas as pl
from jax.experimental.pallas import tpu as pltpu
from jax.experimental.pallas import tpu_sc as plsc

N0, N1, N2 = 10000, 4000, 1000
E1, E2 = 320000, 64000
D = 128
DP = 144
NC, NS = 2, 16
NW = NC * NS
NSEG = 1024
CH = 128


def _sc_aggregate(n_rows, epw, NB):
  mesh = plsc.VectorSubcoreMesh(
      core_axis_name="c", subcore_axis_name="s", num_cores=NC,
      num_subcores=NS)
  zrows = NSEG // NS
  trows = n_rows // NS
  qcap = epw + NB * CH

  @functools.partial(
      pl.kernel,
      out_type=jax.ShapeDtypeStruct((NC, NSEG, DP), jnp.float32),
      mesh=mesh,
      scratch_types=[
          pltpu.VMEM((epw,), jnp.int32),
          pltpu.VMEM((epw,), jnp.int32),
          pltpu.VMEM((qcap,), jnp.int32),
          pltpu.VMEM((qcap,), jnp.int32),
          pltpu.VMEM((NB, CH), jnp.int32),
          pltpu.VMEM((NB, CH), jnp.int32),
          pltpu.VMEM((NB, CH, DP), jnp.float32),
          pltpu.VMEM_SHARED((n_rows, DP), jnp.float32),
          pltpu.VMEM_SHARED((NSEG, DP), jnp.float32),
          pltpu.SemaphoreType.DMA,
          pltpu.SemaphoreType.DMA((NB,)),
          pltpu.SemaphoreType.DMA((NB,)),
      ],
      compiler_params=pltpu.CompilerParams(use_tc_tiling_on_sc=False,
                                           needs_layout_passes=False),
  )
  def agg(table_hbm, edges_hbm, out_hbm, src_v, dst_v, qs, qd, qs2, qd2,
          rows, tbl, acc, semt, semg, sems):
    cid = lax.axis_index("c")
    sid = lax.axis_index("s")
    wid = cid * NS + sid

    tdma = pltpu.make_async_copy(
        table_hbm.at[pl.ds(sid * trows, trows)],
        tbl.at[pl.ds(sid * trows, trows)], semt)
    tdma.start()

    zero = jnp.zeros((16,), jnp.float32)
    for r in range(zrows):
      for k in range(DP // 16):
        rows[0, r, pl.ds(k * 16, 16)] = zero
    pltpu.sync_copy(rows.at[0, pl.ds(0, zrows)],
                    acc.at[pl.ds(sid * zrows, zrows)])

    pltpu.sync_copy(edges_hbm.at[0, pl.ds(wid * epw, epw)], src_v)
    pltpu.sync_copy(edges_hbm.at[1, pl.ds(wid * epw, epw)], dst_v)

    iota = lax.iota(jnp.int32, 16)

    def scan_body(i, qn):
      off = pl.multiple_of(i * 16, 16)
      s = src_v[pl.ds(off, 16)]
      d = dst_v[pl.ds(off, 16)]
      m = d < NSEG
      pos = qn + plsc.cumsum(m.astype(jnp.int32)) - 1
      plsc.store_scatter(qs, [pos], s, mask=m)
      plsc.store_scatter(qd, [pos], d, mask=m)
      return qn + plsc.all_reduce_population_count(m)

    qn_v = lax.fori_loop(0, epw // 16, scan_body, jnp.zeros((16,), jnp.int32))
    for k in range(((NB - 1) * CH) // 16):
      pos = qn_v + k * 16 + iota
      plsc.store_scatter(qs, [pos], jnp.zeros((16,), jnp.int32))
      plsc.store_scatter(qd, [pos], jnp.full((16,), NSEG - 8, jnp.int32))
    qn = jnp.max(qn_v)
    nch = jnp.maximum(qn // CH + 1, NB - 1)
    tdma.wait()
    plsc.subcore_barrier()

    def stage(c, b):
      for k in range(CH // 16):
        pos = c * CH + k * 16 + iota
        qs2[b, pl.ds(k * 16, 16)] = plsc.load_gather(qs, [pos])
        qd2[b, pl.ds(k * 16, 16)] = plsc.load_gather(qd, [pos])

    def gather(b):
      return pltpu.make_async_copy(
          tbl.at[qs2.at[b]], rows.at[b], semg.at[b])

    def scatter_start(b):
      pltpu.async_copy(rows.at[b], acc.at[qd2.at[b]], sems.at[b], add=True)

    def scatter_wait(b):
      pltpu.make_async_copy(rows.at[b], acc.at[qd2.at[b]], sems.at[b]).wait()

    for b in range(NB - 1):
      stage(b, b)
      gather(b).start()

    def body(c, carry):
      b = c % NB
      ob = (c + NB - 1) % NB
      gather(b).wait()
      scatter_start(b)

      @pl.when(c >= 1)
      def _drain():
        scatter_wait(ob)

      stage(c + NB - 1, ob)
      gather(ob).start()
      return carry

    lax.fori_loop(0, nch - (NB - 1), body, 0)

    def tail(k, carry):
      c = nch - (NB - 1) + k
      b = c % NB
      gather(b).wait()
      pltpu.sync_copy(rows.at[b], acc.at[qd2.at[b]], add=True)
      return carry

    lax.fori_loop(0, NB - 1, tail, 0)

    @pl.when(nch > NB - 1)
    def _drain_last_async():
      scatter_wait((nch - NB) % NB)

    plsc.subcore_barrier()
    pltpu.sync_copy(acc.at[pl.ds(sid * zrows, zrows)],
                    out_hbm.at[cid, pl.ds(sid * zrows, zrows)])

  return agg


def _tc_layer1(p, x1, W1l, W1r, b1):
  def body(p_ref, x_ref, wl_ref, wr_ref, b_ref, out_ref):
    ps = p_ref[0, :N2] + p_ref[1, :N2]
    s = ps[:, :D]
    cnt = ps[:, D:D + 1]
    mean = s / jnp.maximum(cnt, 1.0)
    h = mean @ wl_ref[...] + x_ref[...] @ wr_ref[...] + b_ref[...]
    h = jnp.maximum(h, 0.0)
    hp = jnp.concatenate(
        [h, jnp.ones((N2, 1), jnp.float32),
         jnp.zeros((N2, DP - D - 1), jnp.float32)], axis=1)
    out_ref[...] = jnp.concatenate(
        [hp, jnp.zeros((NSEG - N2, DP), jnp.float32)], axis=0)

  return pl.pallas_call(
      body,
      out_shape=jax.ShapeDtypeStruct((NSEG, DP), jnp.float32),
  )(p, x1, W1l, W1r, b1.reshape(1, D))


def _tc_layer2(p, hpad, W2l, W2r, b2):
  def body(p_ref, h_ref, wl_ref, wr_ref, b_ref, out_ref):
    ps = p_ref[0, :N2] + p_ref[1, :N2]
    s = ps[:, :D]
    cnt = ps[:, D:D + 1]
    mean = s / jnp.maximum(cnt, 1.0)
    h = h_ref[:N2, :D]
    z = mean @ wl_ref[...] + h @ wr_ref[...] + b_ref[...]
    m = jnp.max(z, axis=-1, keepdims=True)
    e = jnp.exp(z - m)
    lse = jnp.log(jnp.sum(e, axis=-1, keepdims=True)) + m
    out_ref[...] = z - lse

  return pl.pallas_call(
      body,
      out_shape=jax.ShapeDtypeStruct((N2, D), jnp.float32),
  )(p, hpad, W2l, W2r, b2.reshape(1, D))


def kernel(x, edge_index1, edge_index2, W1l, W1r, b1, W2l, W2r, b2):
  x4 = x[:N1]
  pad = jnp.concatenate(
      [jnp.ones((N1, 1), jnp.float32),
       jnp.zeros((N1, DP - D - 1), jnp.float32)], axis=1)
  xpad = jnp.concatenate([x4, pad], axis=1)

  p1 = _sc_aggregate(N1, E1 // NW, 2)(xpad, edge_index1)
  hpad = _tc_layer1(p1, x[:N2], W1l, W1r, b1)
  p2 = _sc_aggregate(NSEG, E2 // NW, 4)(hpad, edge_index2)
  return _tc_layer2(p2, hpad, W2l, W2r, b2)
